# Initial kernel scaffold; baseline (speedup 1.0000x reference)
import jax
import jax.numpy as jnp
from jax.experimental import pallas as pl


def kernel(x, edge_index, W, b_lin, bias):
    raise NotImplementedError("write your pallas kernel here")


# speedup vs baseline: 18.9806x; 18.9806x over previous
import functools

import jax
import jax.numpy as jnp
from jax import lax
from jax.experimental import pallas as pl
from jax.experimental.pallas import tpu as pltpu
from jax.experimental.pallas import tpu_sc as plsc

N_NODES = 10000
D = 128
E = 320000

NC = 2
NS = 16
NW = NC * NS
EPT = E // NW
K = 128
FULL = EPT // K
TAIL = EPT - FULL * K
ZB = 1000
NZ = N_NODES // ZB

_MESH = plsc.VectorSubcoreMesh(core_axis_name="c", subcore_axis_name="s")


def _deg_body(col_hbm, ones_hbm, zeros_hbm, degp_hbm, idx_v, idx_t, ones_v,
              stage_v, deg_sh):
    cid = lax.axis_index("c")
    sid = lax.axis_index("s")
    base = (cid * NS + sid) * EPT

    @pl.when(sid < NZ)
    def _():
        pltpu.sync_copy(zeros_hbm, stage_v)
        pltpu.sync_copy(stage_v, deg_sh.at[pl.ds(sid * ZB, ZB)])

    pltpu.sync_copy(ones_hbm, ones_v)
    plsc.subcore_barrier()

    def body(c, carry):
        pltpu.sync_copy(col_hbm.at[pl.ds(base + c * K, K)], idx_v)
        pltpu.sync_copy(ones_v, deg_sh.at[idx_v], add=True)
        return carry

    lax.fori_loop(0, FULL, body, 0)
    pltpu.sync_copy(col_hbm.at[pl.ds(base + FULL * K, TAIL)], idx_t)
    pltpu.sync_copy(ones_v.at[pl.ds(0, TAIL)], deg_sh.at[idx_t], add=True)

    plsc.subcore_barrier()

    @pl.when(sid < NZ)
    def _():
        pltpu.sync_copy(deg_sh.at[pl.ds(sid * ZB, ZB)], stage_v)
        pltpu.sync_copy(stage_v,
                        degp_hbm.at[pl.ds(cid * N_NODES + sid * ZB, ZB)])


_deg_call = pl.kernel(
    _deg_body,
    out_type=jax.ShapeDtypeStruct((NC * N_NODES,), jnp.float32),
    mesh=_MESH,
    scratch_types=[
        pltpu.VMEM((K,), jnp.int32),
        pltpu.VMEM((TAIL,), jnp.int32),
        pltpu.VMEM((K,), jnp.float32),
        pltpu.VMEM((ZB,), jnp.float32),
        pltpu.VMEM_SHARED((N_NODES,), jnp.float32),
    ],
)


WB = 200


def _acc_body(g_hbm, row_hbm, col_hbm, zrows_hbm, accp_hbm, ridx_v, cidx_v,
              ridx_t, cidx_t, rows_v, rows_t, zb_v, acc_sh, sem):
    cid = lax.axis_index("c")
    sid = lax.axis_index("s")
    base = (cid * NS + sid) * EPT

    @pl.when(sid < NZ)
    def _():
        pltpu.sync_copy(zrows_hbm, zb_v)

        def zbody(j, carry):
            pltpu.sync_copy(zb_v, acc_sh.at[pl.ds(sid * ZB + j * WB, WB)])
            return carry

        lax.fori_loop(0, ZB // WB, zbody, 0)

    plsc.subcore_barrier()

    def body(c, carry):
        e0 = base + c * K
        pltpu.sync_copy(row_hbm.at[pl.ds(e0, K)], ridx_v)
        pltpu.sync_copy(col_hbm.at[pl.ds(e0, K)], cidx_v)
        pltpu.async_copy(g_hbm.at[ridx_v], rows_v, sem).wait()
        pltpu.sync_copy(rows_v, acc_sh.at[cidx_v], add=True)
        return carry

    lax.fori_loop(0, FULL, body, 0)
    e0 = base + FULL * K
    pltpu.sync_copy(row_hbm.at[pl.ds(e0, TAIL)], ridx_t)
    pltpu.sync_copy(col_hbm.at[pl.ds(e0, TAIL)], cidx_t)
    pltpu.async_copy(g_hbm.at[ridx_t], rows_t, sem).wait()
    pltpu.sync_copy(rows_t, acc_sh.at[cidx_t], add=True)

    plsc.subcore_barrier()

    @pl.when(sid < NZ)
    def _():
        def wbody(j, carry):
            r0 = sid * ZB + j * WB
            pltpu.sync_copy(acc_sh.at[pl.ds(r0, WB)], zb_v)
            pltpu.sync_copy(zb_v, accp_hbm.at[cid, pl.ds(r0, WB)])
            return carry

        lax.fori_loop(0, ZB // WB, wbody, 0)


_acc_call = pl.kernel(
    _acc_body,
    out_type=jax.ShapeDtypeStruct((NC, N_NODES, D), jnp.float32),
    mesh=_MESH,
    scratch_types=[
        pltpu.VMEM((K,), jnp.int32),
        pltpu.VMEM((K,), jnp.int32),
        pltpu.VMEM((TAIL,), jnp.int32),
        pltpu.VMEM((TAIL,), jnp.int32),
        pltpu.VMEM((K, D), jnp.float32),
        pltpu.VMEM((TAIL, D), jnp.float32),
        pltpu.VMEM((WB, D), jnp.float32),
        pltpu.VMEM_SHARED((N_NODES, D), jnp.float32),
        pltpu.SemaphoreType.DMA,
    ],
)


BLK = 1000


def _lin_body(x_ref, w_ref, bl_ref, degp_ref, g_ref):
    deg = degp_ref[:, 0] + degp_ref[:, 1]
    dis = jnp.where(deg > 0.0, lax.rsqrt(deg), 0.0)
    h = jnp.dot(x_ref[...], w_ref[...].T,
                preferred_element_type=jnp.float32) + bl_ref[...]
    g_ref[...] = h * dis[:, None]


_lin_call = pl.pallas_call(
    _lin_body,
    grid=(N_NODES // BLK,),
    in_specs=[
        pl.BlockSpec((BLK, D), lambda i: (i, 0)),
        pl.BlockSpec((D, D), lambda i: (0, 0)),
        pl.BlockSpec((1, D), lambda i: (0, 0)),
        pl.BlockSpec((BLK, NC), lambda i: (i, 0)),
    ],
    out_specs=pl.BlockSpec((BLK, D), lambda i: (i, 0)),
    out_shape=jax.ShapeDtypeStruct((N_NODES, D), jnp.float32),
)


def _out_body(accp_ref, degp_ref, bias_ref, out_ref):
    acc = accp_ref[0] + accp_ref[1]
    deg = degp_ref[:, 0] + degp_ref[:, 1]
    dis = jnp.where(deg > 0.0, lax.rsqrt(deg), 0.0)
    out_ref[...] = jnp.maximum(acc * dis[:, None] + bias_ref[...], 0.0)


_out_call = pl.pallas_call(
    _out_body,
    grid=(N_NODES // BLK,),
    in_specs=[
        pl.BlockSpec((NC, BLK, D), lambda i: (0, i, 0)),
        pl.BlockSpec((BLK, NC), lambda i: (i, 0)),
        pl.BlockSpec((1, D), lambda i: (0, 0)),
    ],
    out_specs=pl.BlockSpec((BLK, D), lambda i: (i, 0)),
    out_shape=jax.ShapeDtypeStruct((N_NODES, D), jnp.float32),
)


@jax.jit
def kernel(x, edge_index, W, b_lin, bias):
    row = edge_index[0]
    col = edge_index[1]
    ones_k = jnp.ones((K,), jnp.float32)
    zeros_n = jnp.zeros((ZB,), jnp.float32)
    zrows = jnp.zeros((WB, D), jnp.float32)

    degp = _deg_call(col, ones_k, zeros_n)
    degp_t = degp.reshape(NC, N_NODES).T
    g = _lin_call(x, W, b_lin.reshape(1, D), degp_t)
    accp = _acc_call(g, row, col, zrows)
    out = _out_call(accp, degp_t, bias.reshape(1, D))
    return out


# speedup vs baseline: 26.4260x; 1.3923x over previous
import functools

import jax
import jax.numpy as jnp
from jax import lax
from jax.experimental import pallas as pl
from jax.experimental.pallas import tpu as pltpu
from jax.experimental.pallas import tpu_sc as plsc

N_NODES = 10000
D = 128
E = 320000

NC = 2
NS = 16
NW = NC * NS
EPT = E // NW
K = 128
FULL = EPT // K
TAIL = EPT - FULL * K
ZB = 1000
NZ = N_NODES // ZB

_MESH = plsc.VectorSubcoreMesh(core_axis_name="c", subcore_axis_name="s")


def _deg_body(col_hbm, ones_hbm, zeros_hbm, degp_hbm, idx_a, idx_b, idx_t,
              ones_v, stage_v, deg_sh, sem_a, sem_b):
    cid = lax.axis_index("c")
    sid = lax.axis_index("s")
    base = (cid * NS + sid) * EPT

    @pl.when(sid < NZ)
    def _():
        pltpu.sync_copy(zeros_hbm, stage_v)
        pltpu.sync_copy(stage_v, deg_sh.at[pl.ds(sid * ZB, ZB)])

    pltpu.sync_copy(ones_hbm, ones_v)
    plsc.subcore_barrier()

    pltpu.sync_copy(col_hbm.at[pl.ds(base, K)], idx_a)

    def body(i, carry):
        a = 2 * i
        pltpu.async_copy(col_hbm.at[pl.ds(base + (a + 1) * K, K)], idx_b,
                         sem_b)
        pltpu.sync_copy(ones_v, deg_sh.at[idx_a], add=True)
        pltpu.make_async_copy(col_hbm.at[pl.ds(base + (a + 1) * K, K)],
                              idx_b, sem_b).wait()
        pltpu.async_copy(col_hbm.at[pl.ds(base + (a + 2) * K, K)], idx_a,
                         sem_a)
        pltpu.sync_copy(ones_v, deg_sh.at[idx_b], add=True)
        pltpu.make_async_copy(col_hbm.at[pl.ds(base + (a + 2) * K, K)],
                              idx_a, sem_a).wait()
        return carry

    lax.fori_loop(0, FULL // 2 - 1, body, 0)
    pltpu.async_copy(col_hbm.at[pl.ds(base + (FULL - 1) * K, K)], idx_b,
                     sem_b)
    pltpu.sync_copy(ones_v, deg_sh.at[idx_a], add=True)
    pltpu.make_async_copy(col_hbm.at[pl.ds(base + (FULL - 1) * K, K)],
                          idx_b, sem_b).wait()
    pltpu.sync_copy(ones_v, deg_sh.at[idx_b], add=True)
    pltpu.sync_copy(col_hbm.at[pl.ds(base + FULL * K, TAIL)], idx_t)
    pltpu.sync_copy(ones_v.at[pl.ds(0, TAIL)], deg_sh.at[idx_t], add=True)

    plsc.subcore_barrier()

    @pl.when(sid < NZ)
    def _():
        pltpu.sync_copy(deg_sh.at[pl.ds(sid * ZB, ZB)], stage_v)
        pltpu.sync_copy(stage_v,
                        degp_hbm.at[pl.ds(cid * N_NODES + sid * ZB, ZB)])


_deg_call = pl.kernel(
    _deg_body,
    out_type=jax.ShapeDtypeStruct((NC * N_NODES,), jnp.float32),
    mesh=_MESH,
    scratch_types=[
        pltpu.VMEM((K,), jnp.int32),
        pltpu.VMEM((K,), jnp.int32),
        pltpu.VMEM((TAIL,), jnp.int32),
        pltpu.VMEM((K,), jnp.float32),
        pltpu.VMEM((ZB,), jnp.float32),
        pltpu.VMEM_SHARED((N_NODES,), jnp.float32),
        pltpu.SemaphoreType.DMA,
        pltpu.SemaphoreType.DMA,
    ],
)


WB = 40


def _acc_body(g_hbm, row_hbm, col_hbm, zrows_hbm, accp_hbm, ridx_a, cidx_a,
              ridx_b, cidx_b, ridx_t, cidx_t, rows_a, rows_b, rows_t, zb_v,
              acc_sh, sem_a, sem_b):
    cid = lax.axis_index("c")
    sid = lax.axis_index("s")
    base = (cid * NS + sid) * EPT

    @pl.when(sid < NZ)
    def _():
        pltpu.sync_copy(zrows_hbm, zb_v)

        def zbody(j, carry):
            pltpu.sync_copy(zb_v, acc_sh.at[pl.ds(sid * ZB + j * WB, WB)])
            return carry

        lax.fori_loop(0, ZB // WB, zbody, 0)

    plsc.subcore_barrier()

    def load_idx(c, ridx, cidx):
        e0 = base + c * K
        pltpu.sync_copy(row_hbm.at[pl.ds(e0, K)], ridx)
        pltpu.sync_copy(col_hbm.at[pl.ds(e0, K)], cidx)

    load_idx(0, ridx_a, cidx_a)
    pltpu.async_copy(g_hbm.at[ridx_a], rows_a, sem_a)
    load_idx(1, ridx_b, cidx_b)

    def body(i, carry):
        a = 2 * i
        pltpu.make_async_copy(g_hbm.at[ridx_a], rows_a, sem_a).wait()
        pltpu.async_copy(g_hbm.at[ridx_b], rows_b, sem_b)
        pltpu.sync_copy(rows_a, acc_sh.at[cidx_a], add=True)
        load_idx(a + 2, ridx_a, cidx_a)
        pltpu.make_async_copy(g_hbm.at[ridx_b], rows_b, sem_b).wait()
        pltpu.async_copy(g_hbm.at[ridx_a], rows_a, sem_a)
        pltpu.sync_copy(rows_b, acc_sh.at[cidx_b], add=True)
        load_idx(a + 3, ridx_b, cidx_b)
        return carry

    lax.fori_loop(0, FULL // 2 - 1, body, 0)
    pltpu.make_async_copy(g_hbm.at[ridx_a], rows_a, sem_a).wait()
    pltpu.async_copy(g_hbm.at[ridx_b], rows_b, sem_b)
    pltpu.sync_copy(rows_a, acc_sh.at[cidx_a], add=True)
    e0 = base + FULL * K
    pltpu.sync_copy(row_hbm.at[pl.ds(e0, TAIL)], ridx_t)
    pltpu.sync_copy(col_hbm.at[pl.ds(e0, TAIL)], cidx_t)
    pltpu.make_async_copy(g_hbm.at[ridx_b], rows_b, sem_b).wait()
    pltpu.async_copy(g_hbm.at[ridx_t], rows_t, sem_a)
    pltpu.sync_copy(rows_b, acc_sh.at[cidx_b], add=True)
    pltpu.make_async_copy(g_hbm.at[ridx_t], rows_t, sem_a).wait()
    pltpu.sync_copy(rows_t, acc_sh.at[cidx_t], add=True)

    plsc.subcore_barrier()

    @pl.when(sid < NZ)
    def _():
        def wbody(j, carry):
            r0 = sid * ZB + j * WB
            pltpu.sync_copy(acc_sh.at[pl.ds(r0, WB)], zb_v)
            pltpu.sync_copy(zb_v, accp_hbm.at[cid, pl.ds(r0, WB)])
            return carry

        lax.fori_loop(0, ZB // WB, wbody, 0)


_acc_call = pl.kernel(
    _acc_body,
    out_type=jax.ShapeDtypeStruct((NC, N_NODES, D), jnp.float32),
    mesh=_MESH,
    scratch_types=[
        pltpu.VMEM((K,), jnp.int32),
        pltpu.VMEM((K,), jnp.int32),
        pltpu.VMEM((K,), jnp.int32),
        pltpu.VMEM((K,), jnp.int32),
        pltpu.VMEM((TAIL,), jnp.int32),
        pltpu.VMEM((TAIL,), jnp.int32),
        pltpu.VMEM((K, D), jnp.float32),
        pltpu.VMEM((K, D), jnp.float32),
        pltpu.VMEM((TAIL, D), jnp.float32),
        pltpu.VMEM((WB, D), jnp.float32),
        pltpu.VMEM_SHARED((N_NODES, D), jnp.float32),
        pltpu.SemaphoreType.DMA,
        pltpu.SemaphoreType.DMA,
    ],
)


BLK = 1000


def _lin_body(x_ref, w_ref, bl_ref, degp_ref, g_ref):
    deg = degp_ref[:, 0] + degp_ref[:, 1]
    dis = jnp.where(deg > 0.0, lax.rsqrt(deg), 0.0)
    h = jnp.dot(x_ref[...], w_ref[...].T,
                preferred_element_type=jnp.float32) + bl_ref[...]
    g_ref[...] = h * dis[:, None]


_lin_call = pl.pallas_call(
    _lin_body,
    grid=(N_NODES // BLK,),
    in_specs=[
        pl.BlockSpec((BLK, D), lambda i: (i, 0)),
        pl.BlockSpec((D, D), lambda i: (0, 0)),
        pl.BlockSpec((1, D), lambda i: (0, 0)),
        pl.BlockSpec((BLK, NC), lambda i: (i, 0)),
    ],
    out_specs=pl.BlockSpec((BLK, D), lambda i: (i, 0)),
    out_shape=jax.ShapeDtypeStruct((N_NODES, D), jnp.float32),
)


def _out_body(accp_ref, degp_ref, bias_ref, out_ref):
    acc = accp_ref[0] + accp_ref[1]
    deg = degp_ref[:, 0] + degp_ref[:, 1]
    dis = jnp.where(deg > 0.0, lax.rsqrt(deg), 0.0)
    out_ref[...] = jnp.maximum(acc * dis[:, None] + bias_ref[...], 0.0)


_out_call = pl.pallas_call(
    _out_body,
    grid=(N_NODES // BLK,),
    in_specs=[
        pl.BlockSpec((NC, BLK, D), lambda i: (0, i, 0)),
        pl.BlockSpec((BLK, NC), lambda i: (i, 0)),
        pl.BlockSpec((1, D), lambda i: (0, 0)),
    ],
    out_specs=pl.BlockSpec((BLK, D), lambda i: (i, 0)),
    out_shape=jax.ShapeDtypeStruct((N_NODES, D), jnp.float32),
)


@jax.jit
def kernel(x, edge_index, W, b_lin, bias):
    row = edge_index[0]
    col = edge_index[1]
    ones_k = jnp.ones((K,), jnp.float32)
    zeros_n = jnp.zeros((ZB,), jnp.float32)
    zrows = jnp.zeros((WB, D), jnp.float32)

    degp = _deg_call(col, ones_k, zeros_n)
    degp_t = degp.reshape(NC, N_NODES).T
    g = _lin_call(x, W, b_lin.reshape(1, D), degp_t)
    accp = _acc_call(g, row, col, zrows)
    out = _out_call(accp, degp_t, bias.reshape(1, D))
    return out
